# Initial kernel scaffold; baseline (speedup 1.0000x reference)
import jax
import jax.numpy as jnp
from jax.experimental import pallas as pl


def kernel(prob, c, z):
    raise NotImplementedError("write your pallas kernel here")


# speedup vs baseline: 188.5673x; 188.5673x over previous
import numpy as np
import jax
import jax.numpy as jnp
from jax.experimental import pallas as pl
from jax.experimental.pallas import tpu as pltpu

_EPS = 1e-12
_KS = 11
_HALF = _KS // 2
_SIGMA = 2.0
_WS = 0.06
_C1 = (0.01 * 1) ** 2
_C2 = (0.03 * 1) ** 2
_H = 32
_W = 32
_HW = _H * _W

_r = np.arange(_KS, dtype=np.float64) - _HALF
_yy, _xx = np.meshgrid(_r, _r, indexing="ij")
_k2 = np.exp(-0.5 * (_xx**2 + _yy**2) / _SIGMA**2)
_k2 = (_k2 / _k2.sum()).astype(np.float32)
_K2 = [[float(_k2[i, j]) for j in range(_KS)] for i in range(_KS)]
_g1 = np.exp(-0.5 * _r**2 / _SIGMA**2)
_g1 = (_g1 / _g1.sum()).astype(np.float32)
_G1 = [float(_g1[i]) for i in range(_KS)]

_MU_B, _MU_C, _HC, _SIG_C, _CC, _CCS, _VALID = range(7)


def _ssim_map(mu_h, mu_c, sig_h, sig_c, sig_hc):
    l = (2.0 * mu_h * mu_c + _C1) / (mu_h * mu_h + mu_c * mu_c + _C1)
    sq = jnp.sqrt(jnp.maximum(sig_h * sig_c, 0.0) + _EPS)
    c_map = (2.0 * sq + _C2) / (sig_h + sig_c + _C2)
    s_map = (2.0 * sig_hc + _C2) / (sq + _C2 + _EPS)
    return l * c_map * s_map


def _marl_kernel(prob_ref, c_ref, h_ref, out_ref, pv, ph, s):
    prob = prob_ref[0]
    c = c_ref[0]
    h = h_ref[0]

    def conv(x):
        pv[:] = jnp.zeros((_H + 2 * _HALF, _W), jnp.float32)
        pv[_HALF:_HALF + _H, :] = x
        tmp = _G1[0] * pv[0:_H, :]
        for i in range(1, _KS):
            tmp = tmp + _G1[i] * pv[i:i + _H, :]
        ph[:] = jnp.zeros((_H, _W + 2 * _HALF), jnp.float32)
        ph[:, _HALF:_HALF + _W] = tmp
        out = _G1[0] * ph[:, 0:_W]
        for j in range(1, _KS):
            out = out + _G1[j] * ph[:, j:j + _W]
        return out

    mu_b = conv(h)
    mu_c = conv(c)
    hvs_c2 = conv(c * c)
    hvs_hc = conv(h * c)
    c_var = conv((c - mu_c) * (c - mu_c))

    sig_c = hvs_c2 - mu_c * mu_c
    cc = jnp.clip(2.0 * jnp.sqrt(c_var + _EPS), 0.0, 1.0)
    sig_h_b = mu_b - mu_b * mu_b
    sig_hc_b = hvs_hc - mu_b * mu_c
    ssim_b = _ssim_map(mu_b, mu_c, sig_h_b, sig_c, sig_hc_b)
    d_b = mu_b - mu_c
    base_map = _WS * (cc * ssim_b + (1.0 - cc)) - d_b * d_b

    s[:] = jnp.zeros((7, _H + 2 * _HALF, _W + 2 * _HALF), jnp.float32)
    sl = slice(_HALF, _HALF + _H)
    s[_MU_B, sl, sl] = mu_b
    s[_MU_C, sl, sl] = mu_c
    s[_HC, sl, sl] = hvs_hc
    s[_SIG_C, sl, sl] = sig_c
    s[_CC, sl, sl] = cc
    s[_CCS, sl, sl] = cc * ssim_b
    s[_VALID, sl, sl] = jnp.ones((_H, _W), jnp.float32)

    delta = 1.0 - 2.0 * h
    w = h + delta * prob

    mse_acc = jnp.zeros((_H, _W), jnp.float32)
    cssim_acc = jnp.zeros((_H, _W), jnp.float32)
    for dy in range(_KS):
        for dx in range(_KS):
            kv = _K2[dy][dx]
            m = s[_MU_B, dy:dy + _H, dx:dx + _W]
            mc = s[_MU_C, dy:dy + _H, dx:dx + _W]
            hc = s[_HC, dy:dy + _H, dx:dx + _W]
            sc = s[_SIG_C, dy:dy + _H, dx:dx + _W]
            ccv = s[_CC, dy:dy + _H, dx:dx + _W]
            ccs = s[_CCS, dy:dy + _H, dx:dx + _W]
            v = s[_VALID, dy:dy + _H, dx:dx + _W]
            dkv = (delta * kv) * v
            mu = m + dkv
            mse_acc = mse_acc + dkv * (2.0 * (m - mc) + dkv)
            sig_h = mu - mu * mu
            sig_hc = (hc + dkv * c) - mu * mc
            ssim_n = _ssim_map(mu, mc, sig_h, sc, sig_hc)
            cssim_acc = cssim_acc + (ccv * ssim_n - ccs)

    d_r = (-mse_acc + _WS * cssim_acc) * (1.0 / _HW)
    out_ref[0] = jnp.sum(base_map + w * d_r, keepdims=True)


def kernel(prob, c, z):
    del z
    b = prob.shape[0]
    u = jax.random.uniform(jax.random.key(42), prob.shape, dtype=prob.dtype)
    h = (u < prob).astype(prob.dtype)

    prob2 = prob.reshape(b, _H, _W)
    c2 = c.reshape(b, _H, _W)
    h2 = h.reshape(b, _H, _W)

    partial = pl.pallas_call(
        _marl_kernel,
        grid=(b,),
        in_specs=[
            pl.BlockSpec((1, _H, _W), lambda i: (i, 0, 0)),
            pl.BlockSpec((1, _H, _W), lambda i: (i, 0, 0)),
            pl.BlockSpec((1, _H, _W), lambda i: (i, 0, 0)),
        ],
        out_specs=pl.BlockSpec((1, 1, 1), lambda i: (i, 0, 0)),
        out_shape=jax.ShapeDtypeStruct((b, 1, 1), jnp.float32),
        scratch_shapes=[
            pltpu.VMEM((_H + 2 * _HALF, _W), jnp.float32),
            pltpu.VMEM((_H, _W + 2 * _HALF), jnp.float32),
            pltpu.VMEM((7, _H + 2 * _HALF, _W + 2 * _HALF), jnp.float32),
        ],
        compiler_params=pltpu.CompilerParams(
            dimension_semantics=("parallel",),
        ),
    )(prob2, c2, h2)

    return -jnp.sum(partial) / (b * _HW)


# speedup vs baseline: 919.3262x; 4.8753x over previous
import numpy as np
import jax
import jax.numpy as jnp
from jax.experimental import pallas as pl
from jax.experimental.pallas import tpu as pltpu

_EPS = 1e-12
_KS = 11
_HALF = _KS // 2
_SIGMA = 2.0
_WS = 0.06
_C1 = (0.01 * 1) ** 2
_C2 = (0.03 * 1) ** 2
_H = 32
_W = 32
_HW = _H * _W

_r = np.arange(_KS, dtype=np.float64) - _HALF
_yy, _xx = np.meshgrid(_r, _r, indexing="ij")
_k2 = np.exp(-0.5 * (_xx**2 + _yy**2) / _SIGMA**2)
_k2 = (_k2 / _k2.sum()).astype(np.float32)
_K2 = [[float(_k2[i, j]) for j in range(_KS)] for i in range(_KS)]
_g1 = np.exp(-0.5 * _r**2 / _SIGMA**2)
_g1 = (_g1 / _g1.sum()).astype(np.float32)
_G1 = [float(_g1[i]) for i in range(_KS)]

(_MU_B, _MU_C, _HC, _SC, _SCC2, _MC2C1, _D2OLD, _CC, _CCS, _VALID) = range(10)
_NSLOT = 10


def _marl_kernel(prob_ref, c_ref, u_ref, out_ref, pv, ph, s, sdx):
    b = prob_ref.shape[0]
    prob = prob_ref[...]
    c = c_ref[...]
    h = jnp.where(u_ref[...] < prob, 1.0, 0.0)

    def conv(x):
        pv[:] = jnp.zeros((b, _H + 2 * _HALF, _W), jnp.float32)
        pv[:, _HALF:_HALF + _H, :] = x
        tmp = _G1[0] * pv[:, 0:_H, :]
        for i in range(1, _KS):
            tmp = tmp + _G1[i] * pv[:, i:i + _H, :]
        ph[:] = jnp.zeros((b, _H, _W + 2 * _HALF), jnp.float32)
        ph[:, :, _HALF:_HALF + _W] = tmp
        out = _G1[0] * ph[:, :, 0:_W]
        for j in range(1, _KS):
            out = out + _G1[j] * ph[:, :, j:j + _W]
        return out

    mu_b = conv(h)
    mu_c = conv(c)
    hvs_c2 = conv(c * c)
    hvs_hc = conv(h * c)
    c_var = conv((c - mu_c) * (c - mu_c))

    sig_c = hvs_c2 - mu_c * mu_c
    cc = jnp.clip(2.0 * jnp.sqrt(c_var + _EPS), 0.0, 1.0)
    sig_h_b = mu_b - mu_b * mu_b
    sig_hc_b = hvs_hc - mu_b * mu_c
    l_b = (2.0 * mu_b * mu_c + _C1) / (mu_b * mu_b + mu_c * mu_c + _C1)
    sq_b = jnp.sqrt(jnp.maximum(sig_h_b * sig_c, 0.0) + _EPS)
    cm_b = (2.0 * sq_b + _C2) / (sig_h_b + sig_c + _C2)
    sm_b = (2.0 * sig_hc_b + _C2) / (sq_b + _C2 + _EPS)
    ssim_b = l_b * cm_b * sm_b
    d_b = mu_b - mu_c
    base_map = _WS * (cc * ssim_b + (1.0 - cc)) - d_b * d_b

    s[:] = jnp.zeros((_NSLOT, b, _H + 2 * _HALF, _W + 2 * _HALF), jnp.float32)
    s[_SCC2] = jnp.full((b, _H + 2 * _HALF, _W + 2 * _HALF), _C2, jnp.float32)
    s[_MC2C1] = jnp.full((b, _H + 2 * _HALF, _W + 2 * _HALF), _C1, jnp.float32)
    sl = slice(_HALF, _HALF + _H)
    s[_MU_B, :, sl, sl] = mu_b
    s[_MU_C, :, sl, sl] = mu_c
    s[_HC, :, sl, sl] = hvs_hc
    s[_SC, :, sl, sl] = sig_c
    s[_SCC2, :, sl, sl] = sig_c + _C2
    s[_MC2C1, :, sl, sl] = mu_c * mu_c + _C1
    s[_D2OLD, :, sl, sl] = 2.0 * d_b
    s[_CC, :, sl, sl] = cc
    s[_CCS, :, sl, sl] = cc * ssim_b
    s[_VALID, :, sl, sl] = jnp.ones((b, _H, _W), jnp.float32)

    delta = 1.0 - 2.0 * h
    w = h + delta * prob

    mse_acc = jnp.zeros((b, _H, _W), jnp.float32)
    cssim_acc = jnp.zeros((b, _H, _W), jnp.float32)
    for dx in range(_KS):
        for t in range(_NSLOT):
            sdx[t] = s[t, :, :, dx:dx + _W]
        for dy in range(_KS):
            kv = _K2[dy][dx]
            m = sdx[_MU_B, :, dy:dy + _H, :]
            mc = sdx[_MU_C, :, dy:dy + _H, :]
            hc = sdx[_HC, :, dy:dy + _H, :]
            sc = sdx[_SC, :, dy:dy + _H, :]
            scc2 = sdx[_SCC2, :, dy:dy + _H, :]
            mc2c1 = sdx[_MC2C1, :, dy:dy + _H, :]
            d2old = sdx[_D2OLD, :, dy:dy + _H, :]
            ccv = sdx[_CC, :, dy:dy + _H, :]
            ccs = sdx[_CCS, :, dy:dy + _H, :]
            v = sdx[_VALID, :, dy:dy + _H, :]
            dkv = (delta * kv) * v
            mu = m + dkv
            mse_acc = mse_acc + dkv * (d2old + dkv)
            mumc = mu * mc
            mu2 = mu * mu
            sig_h = mu - mu2
            sig_hc = (hc + dkv * c) - mumc
            n1 = mumc + mumc + _C1
            d1 = mu2 + mc2c1
            sq = jnp.sqrt(jnp.maximum(sig_h * sc, 0.0) + _EPS)
            d2 = sig_h + scc2
            n2 = sq + sq + _C2
            n3 = sig_hc + sig_hc + _C2
            d3 = sq + (_C2 + _EPS)
            num = ((n1 * n2) * n3) * ccv
            den = (d1 * d2) * d3
            cssim_acc = cssim_acc + (num / den - ccs)

    d_r = (_WS * cssim_acc - mse_acc) * (1.0 / _HW)
    total = base_map + w * d_r
    t0 = jnp.sum(total, axis=0)
    t1 = jnp.sum(t0, axis=0, keepdims=True)
    out_ref[:] = jnp.sum(t1, axis=1, keepdims=True) * (-1.0 / (b * _HW))


def kernel(prob, c, z):
    del z
    b = prob.shape[0]
    u = jax.random.uniform(jax.random.key(42), prob.shape, dtype=prob.dtype)

    out = pl.pallas_call(
        _marl_kernel,
        out_shape=jax.ShapeDtypeStruct((1, 1), jnp.float32),
        scratch_shapes=[
            pltpu.VMEM((b, _H + 2 * _HALF, _W), jnp.float32),
            pltpu.VMEM((b, _H, _W + 2 * _HALF), jnp.float32),
            pltpu.VMEM((_NSLOT, b, _H + 2 * _HALF, _W + 2 * _HALF),
                       jnp.float32),
            pltpu.VMEM((_NSLOT, b, _H + 2 * _HALF, _W), jnp.float32),
        ],
    )(prob.reshape(b, _H, _W), c.reshape(b, _H, _W), u.reshape(b, _H, _W))

    return out.reshape(())


# speedup vs baseline: 1972.1028x; 2.1452x over previous
import numpy as np
import jax
import jax.numpy as jnp
from jax.experimental import pallas as pl
from jax.experimental.pallas import tpu as pltpu

_EPS = 1e-12
_KS = 11
_HALF = _KS // 2
_SIGMA = 2.0
_WS = 0.06
_C1 = (0.01 * 1) ** 2
_C2 = (0.03 * 1) ** 2
_H = 32
_W = 32
_HW = _H * _W
_PAD = 8
_PR = _H + 2 * _PAD
_NL = 128
_L0 = 8
_L1 = 72

_r = np.arange(_KS, dtype=np.float64) - _HALF
_yy, _xx = np.meshgrid(_r, _r, indexing="ij")
_k2 = np.exp(-0.5 * (_xx**2 + _yy**2) / _SIGMA**2)
_k2 = (_k2 / _k2.sum()).astype(np.float32)
_K2 = [[float(_k2[i, j]) for j in range(_KS)] for i in range(_KS)]
_g1 = np.exp(-0.5 * _r**2 / _SIGMA**2)
_g1 = (_g1 / _g1.sum()).astype(np.float32)
_G1 = [float(_g1[i]) for i in range(_KS)]

(_MU_B, _MU_C, _HC, _SC, _SCC2, _MC2C1, _D2OLD, _CC, _CCS, _VALID) = range(10)
_NSLOT = 10


def _psqrt(x):
    return x * jax.lax.rsqrt(x)


def _lroll(x, k):
    k %= _NL
    if k == 0:
        return x
    return jnp.concatenate([x[:, -k:], x[:, :-k]], axis=1)


def _marl_kernel(prob_ref, c_ref, u_ref, out_ref, pk, pv, s, sdx):
    def pack(x0, x1):
        pk[:] = jnp.zeros((_H, _NL), jnp.float32)
        pk[:, _L0:_L0 + _W] = x0
        pk[:, _L1:_L1 + _W] = x1
        return pk[...]

    prob_p = pack(prob_ref[0], prob_ref[1])
    c_p = pack(c_ref[0], c_ref[1])
    ones = jnp.ones((_H, _W), jnp.float32)
    imask = pack(ones, ones)
    h = jnp.where(u_ref[...] < prob_p, 1.0, 0.0)

    def conv(x):
        pv[:] = jnp.zeros((_PR, _NL), jnp.float32)
        pv[_PAD:_PAD + _H, :] = x
        o0 = _PAD - _HALF
        tmp = _G1[0] * pv[o0:o0 + _H, :]
        for i in range(1, _KS):
            tmp = tmp + _G1[i] * pv[o0 + i:o0 + i + _H, :]
        out = _G1[_HALF] * tmp
        for j in range(_KS):
            if j != _HALF:
                out = out + _G1[j] * _lroll(tmp, _HALF - j)
        return out

    mu_b = conv(h)
    mu_c = conv(c_p)
    hvs_c2 = conv(c_p * c_p)
    hvs_hc = conv(h * c_p)
    c_var = conv(imask * ((c_p - mu_c) * (c_p - mu_c)))

    sig_c = hvs_c2 - mu_c * mu_c
    cc = imask * jnp.clip(2.0 * _psqrt(c_var + _EPS), 0.0, 1.0)
    sig_h_b = mu_b - mu_b * mu_b
    sig_hc_b = hvs_hc - mu_b * mu_c
    l_b = (2.0 * mu_b * mu_c + _C1) / (mu_b * mu_b + mu_c * mu_c + _C1)
    sq_b = _psqrt(jnp.maximum(sig_h_b * sig_c, 0.0) + _EPS)
    cm_b = (2.0 * sq_b + _C2) / (sig_h_b + sig_c + _C2)
    sm_b = (2.0 * sig_hc_b + _C2) / (sq_b + _C2 + _EPS)
    ssim_b = l_b * cm_b * sm_b
    d_b = mu_b - mu_c
    base_map = imask * (_WS * (cc * ssim_b + (1.0 - cc)) - d_b * d_b)

    s[:] = jnp.zeros((_NSLOT, _PR, _NL), jnp.float32)
    s[_SCC2] = jnp.full((_PR, _NL), _C2, jnp.float32)
    s[_MC2C1] = jnp.full((_PR, _NL), _C1, jnp.float32)
    rows = slice(_PAD, _PAD + _H)
    s[_MU_B, rows, :] = mu_b
    s[_MU_C, rows, :] = mu_c
    s[_HC, rows, :] = hvs_hc
    s[_SC, rows, :] = sig_c
    s[_SCC2, rows, :] = sig_c + _C2
    s[_MC2C1, rows, :] = mu_c * mu_c + _C1
    s[_D2OLD, rows, :] = 2.0 * d_b
    s[_CC, rows, :] = cc
    s[_CCS, rows, :] = cc * ssim_b
    s[_VALID, rows, :] = imask

    delta = 1.0 - 2.0 * h
    w = h + delta * prob_p

    mse_acc = jnp.zeros((_H, _NL), jnp.float32)
    cssim_acc = jnp.zeros((_H, _NL), jnp.float32)
    for dx in range(_KS):
        for t in range(_NSLOT):
            sdx[t] = _lroll(s[t], _HALF - dx)
        for dy in range(_KS):
            kv = _K2[dy][dx]
            y0 = _PAD - _HALF + dy
            m = sdx[_MU_B, y0:y0 + _H, :]
            mc = sdx[_MU_C, y0:y0 + _H, :]
            hc = sdx[_HC, y0:y0 + _H, :]
            sc = sdx[_SC, y0:y0 + _H, :]
            scc2 = sdx[_SCC2, y0:y0 + _H, :]
            mc2c1 = sdx[_MC2C1, y0:y0 + _H, :]
            d2old = sdx[_D2OLD, y0:y0 + _H, :]
            ccv = sdx[_CC, y0:y0 + _H, :]
            ccs = sdx[_CCS, y0:y0 + _H, :]
            v = sdx[_VALID, y0:y0 + _H, :]
            dkv = (delta * kv) * v
            mu = m + dkv
            mse_acc = mse_acc + dkv * (d2old + dkv)
            mumc = mu * mc
            mu2 = mu * mu
            sig_h = mu - mu2
            sig_hc = (hc + dkv * c_p) - mumc
            n1 = mumc + mumc + _C1
            d1 = mu2 + mc2c1
            sq = _psqrt(jnp.maximum(sig_h * sc, 0.0) + _EPS)
            d2 = sig_h + scc2
            n2 = sq + sq + _C2
            n3 = sig_hc + sig_hc + _C2
            d3 = sq + (_C2 + _EPS)
            num = ((n1 * n2) * n3) * ccv
            den = (d1 * d2) * d3
            cssim_acc = cssim_acc + (num / den - ccs)

    d_r = (_WS * cssim_acc - mse_acc) * (1.0 / _HW)
    total = base_map + w * d_r
    t1 = jnp.sum(total, axis=0, keepdims=True)
    out_ref[:] = jnp.sum(t1, axis=1, keepdims=True) * (-1.0 / (2 * _HW))


def kernel(prob, c, z):
    del z
    b = prob.shape[0]
    with jax.ensure_compile_time_eval():
        u = np.asarray(
            jax.random.uniform(jax.random.key(42), prob.shape,
                               dtype=prob.dtype)
        ).reshape(b, _H, _W)
    up = np.ones((_H, _NL), np.float32)
    up[:, _L0:_L0 + _W] = u[0]
    up[:, _L1:_L1 + _W] = u[1]

    out = pl.pallas_call(
        _marl_kernel,
        out_shape=jax.ShapeDtypeStruct((1, 1), jnp.float32),
        scratch_shapes=[
            pltpu.VMEM((_H, _NL), jnp.float32),
            pltpu.VMEM((_PR, _NL), jnp.float32),
            pltpu.VMEM((_NSLOT, _PR, _NL), jnp.float32),
            pltpu.VMEM((_NSLOT, _PR, _NL), jnp.float32),
        ],
    )(prob.reshape(b, _H, _W), c.reshape(b, _H, _W), jnp.asarray(up))

    return out.reshape(())


# speedup vs baseline: 1981.5815x; 1.0048x over previous
import numpy as np
import jax
import jax.numpy as jnp
from jax.experimental import pallas as pl
from jax.experimental.pallas import tpu as pltpu

_EPS = 1e-12
_KS = 11
_HALF = _KS // 2
_SIGMA = 2.0
_WS = 0.06
_C1 = (0.01 * 1) ** 2
_C2 = (0.03 * 1) ** 2
_H = 32
_W = 32
_HW = _H * _W
_PAD = 8
_PR = _H + 2 * _PAD
_NL = 128
_L0 = 8
_L1 = 72

_r = np.arange(_KS, dtype=np.float64) - _HALF
_yy, _xx = np.meshgrid(_r, _r, indexing="ij")
_k2 = np.exp(-0.5 * (_xx**2 + _yy**2) / _SIGMA**2)
_k2 = (_k2 / _k2.sum()).astype(np.float32)
_K2 = [[float(_k2[i, j]) for j in range(_KS)] for i in range(_KS)]
_g1 = np.exp(-0.5 * _r**2 / _SIGMA**2)
_g1 = (_g1 / _g1.sum()).astype(np.float32)
_G1 = [float(_g1[i]) for i in range(_KS)]

(_MU_B, _MU_C, _HC, _SC, _SCC2, _MC2C1, _D2OLD, _CC, _CCS, _VALID) = range(10)
_NSLOT = 10


def _psqrt(x):
    return x * jax.lax.rsqrt(x)


def _np_threefry2x32(k1, k2, x0, x1):
    rot_a = (13, 15, 26, 6)
    rot_b = (17, 29, 16, 24)

    def rl(x, r):
        return ((x << np.uint32(r)) | (x >> np.uint32(32 - r))).astype(
            np.uint32)

    def rounds(x, rs):
        for r in rs:
            x[0] = (x[0] + x[1]).astype(np.uint32)
            x[1] = x[0] ^ rl(x[1], r)
        return x

    ks = [k1, k2, np.uint32(k1 ^ k2 ^ np.uint32(0x1BD11BDA))]
    x = [(x0 + ks[0]).astype(np.uint32), (x1 + ks[1]).astype(np.uint32)]
    sched = [(rot_a, 1, 2), (rot_b, 2, 0), (rot_a, 0, 1), (rot_b, 1, 2),
             (rot_a, 2, 0)]
    for i, (rs, a, b) in enumerate(sched):
        x = rounds(x, rs)
        x[0] = (x[0] + ks[a]).astype(np.uint32)
        x[1] = (x[1] + ks[b] + np.uint32(i + 1)).astype(np.uint32)
    return x


def _np_uniform(seed, shape):
    n = int(np.prod(shape))
    hi = np.zeros(n, dtype=np.uint32)
    lo = np.arange(n, dtype=np.uint32)
    b = _np_threefry2x32(np.uint32(seed >> 32), np.uint32(seed & 0xFFFFFFFF),
                         hi, lo)
    bits = (b[0] ^ b[1]).astype(np.uint32)
    fb = ((bits >> np.uint32(9)) | np.uint32(0x3F800000)).view(
        np.float32) - np.float32(1.0)
    return np.maximum(np.float32(0.0), fb).reshape(shape)


def _lroll(x, k):
    k %= _NL
    if k == 0:
        return x
    return jnp.concatenate([x[:, -k:], x[:, :-k]], axis=1)


def _marl_kernel(prob_ref, c_ref, u_ref, out_ref, pk, pv, s, sdx):
    def pack(x0, x1):
        pk[:] = jnp.zeros((_H, _NL), jnp.float32)
        pk[:, _L0:_L0 + _W] = x0
        pk[:, _L1:_L1 + _W] = x1
        return pk[...]

    prob_p = pack(prob_ref[0], prob_ref[1])
    c_p = pack(c_ref[0], c_ref[1])
    ones = jnp.ones((_H, _W), jnp.float32)
    imask = pack(ones, ones)
    h = jnp.where(u_ref[...] < prob_p, 1.0, 0.0)

    def conv(x):
        pv[:] = jnp.zeros((_PR, _NL), jnp.float32)
        pv[_PAD:_PAD + _H, :] = x
        o0 = _PAD - _HALF
        tmp = _G1[0] * pv[o0:o0 + _H, :]
        for i in range(1, _KS):
            tmp = tmp + _G1[i] * pv[o0 + i:o0 + i + _H, :]
        out = _G1[_HALF] * tmp
        for j in range(_KS):
            if j != _HALF:
                out = out + _G1[j] * _lroll(tmp, _HALF - j)
        return out

    mu_b = conv(h)
    mu_c = conv(c_p)
    hvs_c2 = conv(c_p * c_p)
    hvs_hc = conv(h * c_p)
    c_var = conv(imask * ((c_p - mu_c) * (c_p - mu_c)))

    sig_c = hvs_c2 - mu_c * mu_c
    cc = imask * jnp.clip(2.0 * _psqrt(c_var + _EPS), 0.0, 1.0)
    sig_h_b = mu_b - mu_b * mu_b
    sig_hc_b = hvs_hc - mu_b * mu_c
    l_b = (2.0 * mu_b * mu_c + _C1) / (mu_b * mu_b + mu_c * mu_c + _C1)
    sq_b = _psqrt(jnp.maximum(sig_h_b * sig_c, 0.0) + _EPS)
    cm_b = (2.0 * sq_b + _C2) / (sig_h_b + sig_c + _C2)
    sm_b = (2.0 * sig_hc_b + _C2) / (sq_b + _C2 + _EPS)
    ssim_b = l_b * cm_b * sm_b
    d_b = mu_b - mu_c
    base_map = imask * (_WS * (cc * ssim_b + (1.0 - cc)) - d_b * d_b)

    s[:] = jnp.zeros((_NSLOT, _PR, _NL), jnp.float32)
    s[_SCC2] = jnp.full((_PR, _NL), _C2, jnp.float32)
    s[_MC2C1] = jnp.full((_PR, _NL), _C1, jnp.float32)
    rows = slice(_PAD, _PAD + _H)
    s[_MU_B, rows, :] = mu_b
    s[_MU_C, rows, :] = mu_c
    s[_HC, rows, :] = hvs_hc
    s[_SC, rows, :] = sig_c
    s[_SCC2, rows, :] = sig_c + _C2
    s[_MC2C1, rows, :] = mu_c * mu_c + _C1
    s[_D2OLD, rows, :] = 2.0 * d_b
    s[_CC, rows, :] = cc
    s[_CCS, rows, :] = cc * ssim_b
    s[_VALID, rows, :] = imask

    delta = 1.0 - 2.0 * h
    w = h + delta * prob_p

    mse_acc = jnp.zeros((_H, _NL), jnp.float32)
    cssim_acc = jnp.zeros((_H, _NL), jnp.float32)
    for dx in range(_KS):
        for t in range(_NSLOT):
            sdx[t] = _lroll(s[t], _HALF - dx)
        for dy in range(_KS):
            kv = _K2[dy][dx]
            y0 = _PAD - _HALF + dy
            m = sdx[_MU_B, y0:y0 + _H, :]
            mc = sdx[_MU_C, y0:y0 + _H, :]
            hc = sdx[_HC, y0:y0 + _H, :]
            sc = sdx[_SC, y0:y0 + _H, :]
            scc2 = sdx[_SCC2, y0:y0 + _H, :]
            mc2c1 = sdx[_MC2C1, y0:y0 + _H, :]
            d2old = sdx[_D2OLD, y0:y0 + _H, :]
            ccv = sdx[_CC, y0:y0 + _H, :]
            ccs = sdx[_CCS, y0:y0 + _H, :]
            v = sdx[_VALID, y0:y0 + _H, :]
            dkv = (delta * kv) * v
            mu = m + dkv
            mse_acc = mse_acc + dkv * (d2old + dkv)
            mumc = mu * mc
            mu2 = mu * mu
            sig_h = mu - mu2
            sig_hc = (hc + dkv * c_p) - mumc
            n1 = mumc + mumc + _C1
            d1 = mu2 + mc2c1
            sq = _psqrt(jnp.maximum(sig_h * sc, 0.0) + _EPS)
            d2 = sig_h + scc2
            n2 = sq + sq + _C2
            n3 = sig_hc + sig_hc + _C2
            d3 = sq + (_C2 + _EPS)
            num = ((n1 * n2) * n3) * ccv
            den = (d1 * d2) * d3
            cssim_acc = cssim_acc + (num / den - ccs)

    d_r = (_WS * cssim_acc - mse_acc) * (1.0 / _HW)
    total = base_map + w * d_r
    t1 = jnp.sum(total, axis=0, keepdims=True)
    out_ref[:] = jnp.sum(t1, axis=1, keepdims=True) * (-1.0 / (2 * _HW))


def kernel(prob, c, z):
    del z
    b = prob.shape[0]
    u = _np_uniform(42, prob.shape).reshape(b, _H, _W)
    up = np.ones((_H, _NL), np.float32)
    up[:, _L0:_L0 + _W] = u[0]
    up[:, _L1:_L1 + _W] = u[1]

    out = pl.pallas_call(
        _marl_kernel,
        out_shape=jax.ShapeDtypeStruct((1, 1), jnp.float32),
        scratch_shapes=[
            pltpu.VMEM((_H, _NL), jnp.float32),
            pltpu.VMEM((_PR, _NL), jnp.float32),
            pltpu.VMEM((_NSLOT, _PR, _NL), jnp.float32),
            pltpu.VMEM((_NSLOT, _PR, _NL), jnp.float32),
        ],
    )(prob.reshape(b, _H, _W), c.reshape(b, _H, _W), jnp.asarray(up))

    return out.reshape(())


# speedup vs baseline: 2071.1772x; 1.0452x over previous
import numpy as np
import jax
import jax.numpy as jnp
from jax.experimental import pallas as pl
from jax.experimental.pallas import tpu as pltpu

_EPS = 1e-12
_KS = 11
_HALF = _KS // 2
_SIGMA = 2.0
_WS = 0.06
_C1 = (0.01 * 1) ** 2
_C2 = (0.03 * 1) ** 2
_H = 32
_W = 32
_HW = _H * _W
_PAD = 8
_PR = _H + 2 * _PAD
_NL = 128
_L0 = 8
_L1 = 72

_r = np.arange(_KS, dtype=np.float64) - _HALF
_yy, _xx = np.meshgrid(_r, _r, indexing="ij")
_k2 = np.exp(-0.5 * (_xx**2 + _yy**2) / _SIGMA**2)
_k2 = (_k2 / _k2.sum()).astype(np.float32)
_K2 = [[float(_k2[i, j]) for j in range(_KS)] for i in range(_KS)]
_g1 = np.exp(-0.5 * _r**2 / _SIGMA**2)
_g1 = (_g1 / _g1.sum()).astype(np.float32)
_G1 = [float(_g1[i]) for i in range(_KS)]

_G1SQ = [g * g for g in _G1]
_BOX = [1.0] * _KS

(_MU_B, _MU_C, _HC, _SC, _SCC2, _MC2C1, _CC, _VALID) = range(8)
_NSLOT = 8


def _psqrt(x):
    return x * jax.lax.rsqrt(x)


def _np_threefry2x32(k1, k2, x0, x1):
    rot_a = (13, 15, 26, 6)
    rot_b = (17, 29, 16, 24)

    def rl(x, r):
        return ((x << np.uint32(r)) | (x >> np.uint32(32 - r))).astype(
            np.uint32)

    def rounds(x, rs):
        for r in rs:
            x[0] = (x[0] + x[1]).astype(np.uint32)
            x[1] = x[0] ^ rl(x[1], r)
        return x

    ks = [k1, k2, np.uint32(k1 ^ k2 ^ np.uint32(0x1BD11BDA))]
    x = [(x0 + ks[0]).astype(np.uint32), (x1 + ks[1]).astype(np.uint32)]
    sched = [(rot_a, 1, 2), (rot_b, 2, 0), (rot_a, 0, 1), (rot_b, 1, 2),
             (rot_a, 2, 0)]
    for i, (rs, a, b) in enumerate(sched):
        x = rounds(x, rs)
        x[0] = (x[0] + ks[a]).astype(np.uint32)
        x[1] = (x[1] + ks[b] + np.uint32(i + 1)).astype(np.uint32)
    return x


def _np_uniform(seed, shape):
    n = int(np.prod(shape))
    hi = np.zeros(n, dtype=np.uint32)
    lo = np.arange(n, dtype=np.uint32)
    b = _np_threefry2x32(np.uint32(seed >> 32), np.uint32(seed & 0xFFFFFFFF),
                         hi, lo)
    bits = (b[0] ^ b[1]).astype(np.uint32)
    fb = ((bits >> np.uint32(9)) | np.uint32(0x3F800000)).view(
        np.float32) - np.float32(1.0)
    return np.maximum(np.float32(0.0), fb).reshape(shape)


def _lroll(x, k):
    k %= _NL
    if k == 0:
        return x
    return jnp.concatenate([x[:, -k:], x[:, :-k]], axis=1)


def _marl_kernel(prob_ref, c_ref, u_ref, out_ref, pk, pv, s, sdx):
    def pack(x0, x1):
        pk[:] = jnp.zeros((_H, _NL), jnp.float32)
        pk[:, _L0:_L0 + _W] = x0
        pk[:, _L1:_L1 + _W] = x1
        return pk[...]

    prob_p = pack(prob_ref[0], prob_ref[1])
    c_p = pack(c_ref[0], c_ref[1])
    ones = jnp.ones((_H, _W), jnp.float32)
    imask = pack(ones, ones)
    h = jnp.where(u_ref[...] < prob_p, 1.0, 0.0)

    def conv(x, g=_G1):
        pv[:] = jnp.zeros((_PR, _NL), jnp.float32)
        pv[_PAD:_PAD + _H, :] = x
        o0 = _PAD - _HALF
        tmp = g[0] * pv[o0:o0 + _H, :]
        for i in range(1, _KS):
            tmp = tmp + g[i] * pv[o0 + i:o0 + i + _H, :]
        out = g[_HALF] * tmp
        for j in range(_KS):
            if j != _HALF:
                out = out + g[j] * _lroll(tmp, _HALF - j)
        return out

    mu_b = conv(h)
    mu_c = conv(c_p)
    hvs_c2 = conv(c_p * c_p)
    hvs_hc = conv(h * c_p)
    c_var = conv(imask * ((c_p - mu_c) * (c_p - mu_c)))

    sig_c = hvs_c2 - mu_c * mu_c
    cc = imask * jnp.clip(2.0 * _psqrt(c_var + _EPS), 0.0, 1.0)
    sig_h_b = mu_b - mu_b * mu_b
    sig_hc_b = hvs_hc - mu_b * mu_c
    l_b = (2.0 * mu_b * mu_c + _C1) / (mu_b * mu_b + mu_c * mu_c + _C1)
    sq_b = _psqrt(jnp.maximum(sig_h_b * sig_c, 0.0) + _EPS)
    cm_b = (2.0 * sq_b + _C2) / (sig_h_b + sig_c + _C2)
    sm_b = (2.0 * sig_hc_b + _C2) / (sq_b + _C2 + _EPS)
    ssim_b = l_b * cm_b * sm_b
    d_b = mu_b - mu_c
    base_map = imask * (_WS * (cc * ssim_b + (1.0 - cc)) - d_b * d_b)

    conv_db2 = 2.0 * conv(imask * d_b)
    k2sum = conv(imask, _G1SQ)
    box_ccs = conv(cc * ssim_b, _BOX)

    s[:] = jnp.zeros((_NSLOT, _PR, _NL), jnp.float32)
    s[_SCC2] = jnp.full((_PR, _NL), _C2, jnp.float32)
    s[_MC2C1] = jnp.full((_PR, _NL), _C1, jnp.float32)
    rows = slice(_PAD, _PAD + _H)
    s[_MU_B, rows, :] = mu_b
    s[_MU_C, rows, :] = mu_c
    s[_HC, rows, :] = hvs_hc
    s[_SC, rows, :] = sig_c
    s[_SCC2, rows, :] = sig_c + _C2
    s[_MC2C1, rows, :] = mu_c * mu_c + _C1
    s[_CC, rows, :] = cc
    s[_VALID, rows, :] = imask

    delta = 1.0 - 2.0 * h
    w = h + delta * prob_p
    mse_acc = delta * conv_db2 + k2sum

    cssim_acc = jnp.zeros((_H, _NL), jnp.float32)
    for dx in range(_KS):
        for t in range(_NSLOT):
            sdx[t] = _lroll(s[t], _HALF - dx)
        for dy in range(_KS):
            kv = _K2[dy][dx]
            y0 = _PAD - _HALF + dy
            m = sdx[_MU_B, y0:y0 + _H, :]
            mc = sdx[_MU_C, y0:y0 + _H, :]
            hc = sdx[_HC, y0:y0 + _H, :]
            sc = sdx[_SC, y0:y0 + _H, :]
            scc2 = sdx[_SCC2, y0:y0 + _H, :]
            mc2c1 = sdx[_MC2C1, y0:y0 + _H, :]
            ccv = sdx[_CC, y0:y0 + _H, :]
            v = sdx[_VALID, y0:y0 + _H, :]
            dkv = (delta * kv) * v
            mu = m + dkv
            mumc = mu * mc
            mu2 = mu * mu
            sig_h = mu - mu2
            sig_hc = (hc + dkv * c_p) - mumc
            n1 = mumc + mumc + _C1
            d1 = mu2 + mc2c1
            sq = _psqrt(jnp.maximum(sig_h * sc, 0.0) + _EPS)
            d2 = sig_h + scc2
            n2 = sq + sq + _C2
            n3 = sig_hc + sig_hc + _C2
            d3 = sq + (_C2 + _EPS)
            num = ((n1 * n2) * n3) * ccv
            den = (d1 * d2) * d3
            cssim_acc = cssim_acc + num / den

    d_r = (_WS * (cssim_acc - box_ccs) - mse_acc) * (1.0 / _HW)
    total = base_map + w * d_r
    t1 = jnp.sum(total, axis=0, keepdims=True)
    out_ref[:] = jnp.sum(t1, axis=1, keepdims=True) * (-1.0 / (2 * _HW))


def kernel(prob, c, z):
    del z
    b = prob.shape[0]
    u = _np_uniform(42, prob.shape).reshape(b, _H, _W)
    up = np.ones((_H, _NL), np.float32)
    up[:, _L0:_L0 + _W] = u[0]
    up[:, _L1:_L1 + _W] = u[1]

    out = pl.pallas_call(
        _marl_kernel,
        out_shape=jax.ShapeDtypeStruct((1, 1), jnp.float32),
        scratch_shapes=[
            pltpu.VMEM((_H, _NL), jnp.float32),
            pltpu.VMEM((_PR, _NL), jnp.float32),
            pltpu.VMEM((_NSLOT, _PR, _NL), jnp.float32),
            pltpu.VMEM((_NSLOT, _PR, _NL), jnp.float32),
        ],
    )(prob.reshape(b, _H, _W), c.reshape(b, _H, _W), jnp.asarray(up))

    return out.reshape(())


# speedup vs baseline: 2197.3299x; 1.0609x over previous
import numpy as np
import jax
import jax.numpy as jnp
from jax.experimental import pallas as pl
from jax.experimental.pallas import tpu as pltpu

_EPS = 1e-12
_KS = 11
_HALF = _KS // 2
_SIGMA = 2.0
_WS = 0.06
_C1 = (0.01 * 1) ** 2
_C2 = (0.03 * 1) ** 2
_H = 32
_W = 32
_HW = _H * _W
_PAD = 8
_PR = _H + 2 * _PAD
_NL = 128
_L0 = 8
_L1 = 72

_r = np.arange(_KS, dtype=np.float64) - _HALF
_yy, _xx = np.meshgrid(_r, _r, indexing="ij")
_k2 = np.exp(-0.5 * (_xx**2 + _yy**2) / _SIGMA**2)
_k2 = (_k2 / _k2.sum()).astype(np.float32)
_K2 = [[float(_k2[i, j]) for j in range(_KS)] for i in range(_KS)]
_g1 = np.exp(-0.5 * _r**2 / _SIGMA**2)
_g1 = (_g1 / _g1.sum()).astype(np.float32)
_G1 = [float(_g1[i]) for i in range(_KS)]

_G1SQ = [g * g for g in _G1]
_BOX = [1.0] * _KS

(_MU_B, _MU_C, _HC, _SC, _SCC2, _MC2C1, _CC) = range(7)
_NSLOT = 7


def _psqrt(x):
    return x * jax.lax.rsqrt(x)


def _np_threefry2x32(k1, k2, x0, x1):
    rot_a = (13, 15, 26, 6)
    rot_b = (17, 29, 16, 24)

    def rl(x, r):
        return ((x << np.uint32(r)) | (x >> np.uint32(32 - r))).astype(
            np.uint32)

    def rounds(x, rs):
        for r in rs:
            x[0] = (x[0] + x[1]).astype(np.uint32)
            x[1] = x[0] ^ rl(x[1], r)
        return x

    ks = [k1, k2, np.uint32(k1 ^ k2 ^ np.uint32(0x1BD11BDA))]
    x = [(x0 + ks[0]).astype(np.uint32), (x1 + ks[1]).astype(np.uint32)]
    sched = [(rot_a, 1, 2), (rot_b, 2, 0), (rot_a, 0, 1), (rot_b, 1, 2),
             (rot_a, 2, 0)]
    for i, (rs, a, b) in enumerate(sched):
        x = rounds(x, rs)
        x[0] = (x[0] + ks[a]).astype(np.uint32)
        x[1] = (x[1] + ks[b] + np.uint32(i + 1)).astype(np.uint32)
    return x


def _np_uniform(seed, shape):
    n = int(np.prod(shape))
    hi = np.zeros(n, dtype=np.uint32)
    lo = np.arange(n, dtype=np.uint32)
    b = _np_threefry2x32(np.uint32(seed >> 32), np.uint32(seed & 0xFFFFFFFF),
                         hi, lo)
    bits = (b[0] ^ b[1]).astype(np.uint32)
    fb = ((bits >> np.uint32(9)) | np.uint32(0x3F800000)).view(
        np.float32) - np.float32(1.0)
    return np.maximum(np.float32(0.0), fb).reshape(shape)


def _lroll(x, k):
    k %= _NL
    if k == 0:
        return x
    return jnp.concatenate([x[:, -k:], x[:, :-k]], axis=1)


def _marl_kernel(prob_ref, c_ref, u_ref, out_ref, pk, pv, s, sdx):
    def pack(x0, x1):
        pk[:] = jnp.zeros((_H, _NL), jnp.float32)
        pk[:, _L0:_L0 + _W] = x0
        pk[:, _L1:_L1 + _W] = x1
        return pk[...]

    prob_p = pack(prob_ref[0], prob_ref[1])
    c_p = pack(c_ref[0], c_ref[1])
    ones = jnp.ones((_H, _W), jnp.float32)
    imask = pack(ones, ones)
    h = jnp.where(u_ref[...] < prob_p, 1.0, 0.0)

    def conv(x, g=_G1):
        pv[:] = jnp.zeros((_PR, _NL), jnp.float32)
        pv[_PAD:_PAD + _H, :] = x
        o0 = _PAD - _HALF
        tmp = g[0] * pv[o0:o0 + _H, :]
        for i in range(1, _KS):
            tmp = tmp + g[i] * pv[o0 + i:o0 + i + _H, :]
        out = g[_HALF] * tmp
        for j in range(_KS):
            if j != _HALF:
                out = out + g[j] * _lroll(tmp, _HALF - j)
        return out

    mu_b = conv(h)
    mu_c = conv(c_p)
    hvs_c2 = conv(c_p * c_p)
    hvs_hc = conv(h * c_p)
    c_var = conv(imask * ((c_p - mu_c) * (c_p - mu_c)))

    sig_c = hvs_c2 - mu_c * mu_c
    cc = imask * jnp.clip(2.0 * _psqrt(c_var + _EPS), 0.0, 1.0)
    sig_h_b = mu_b - mu_b * mu_b
    sig_hc_b = hvs_hc - mu_b * mu_c
    l_b = (2.0 * mu_b * mu_c + _C1) / (mu_b * mu_b + mu_c * mu_c + _C1)
    sq_b = _psqrt(jnp.maximum(sig_h_b * sig_c, 0.0) + _EPS)
    cm_b = (2.0 * sq_b + _C2) / (sig_h_b + sig_c + _C2)
    sm_b = (2.0 * sig_hc_b + _C2) / (sq_b + _C2 + _EPS)
    ssim_b = l_b * cm_b * sm_b
    d_b = mu_b - mu_c
    base_map = imask * (_WS * (cc * ssim_b + (1.0 - cc)) - d_b * d_b)

    conv_db2 = 2.0 * conv(imask * d_b)
    k2sum = conv(imask, _G1SQ)
    box_ccs = conv(cc * ssim_b, _BOX)

    s[:] = jnp.zeros((_NSLOT, _PR, _NL), jnp.float32)
    s[_SCC2] = jnp.full((_PR, _NL), _C2, jnp.float32)
    s[_MC2C1] = jnp.full((_PR, _NL), _C1, jnp.float32)
    rows = slice(_PAD, _PAD + _H)
    s[_MU_B, rows, :] = mu_b
    s[_MU_C, rows, :] = mu_c
    s[_HC, rows, :] = hvs_hc
    s[_SC, rows, :] = sig_c
    s[_SCC2, rows, :] = sig_c + _C2
    s[_MC2C1, rows, :] = mu_c * mu_c + _C1
    s[_CC, rows, :] = cc

    delta = 1.0 - 2.0 * h
    w = h + delta * prob_p
    mse_acc = delta * conv_db2 + k2sum

    cssim_acc = jnp.zeros((_H, _NL), jnp.float32)
    for dx in range(_KS):
        for t in range(_NSLOT):
            sdx[t] = _lroll(s[t], _HALF - dx)
        for dy in range(_KS):
            kv = _K2[dy][dx]
            y0 = _PAD - _HALF + dy
            m = sdx[_MU_B, y0:y0 + _H, :]
            mc = sdx[_MU_C, y0:y0 + _H, :]
            hc = sdx[_HC, y0:y0 + _H, :]
            sc = sdx[_SC, y0:y0 + _H, :]
            scc2 = sdx[_SCC2, y0:y0 + _H, :]
            mc2c1 = sdx[_MC2C1, y0:y0 + _H, :]
            ccv = sdx[_CC, y0:y0 + _H, :]
            dkv = delta * kv
            mu = m + dkv
            mumc = mu * mc
            mu2 = mu * mu
            sig_h = mu - mu2
            sig_hc = (hc + dkv * c_p) - mumc
            n1 = mumc + mumc + _C1
            d1 = mu2 + mc2c1
            sq = _psqrt(jnp.maximum(sig_h * sc, 0.0) + _EPS)
            d2 = sig_h + scc2
            n2 = sq + sq + _C2
            n3 = sig_hc + sig_hc + _C2
            d3 = sq + (_C2 + _EPS)
            num = ((n1 * n2) * n3) * ccv
            den = (d1 * d2) * d3
            cssim_acc = cssim_acc + num / den

    d_r = (_WS * (cssim_acc - box_ccs) - mse_acc) * (1.0 / _HW)
    total = base_map + w * d_r
    t1 = jnp.sum(total, axis=0, keepdims=True)
    out_ref[:] = jnp.sum(t1, axis=1, keepdims=True) * (-1.0 / (2 * _HW))


def kernel(prob, c, z):
    del z
    b = prob.shape[0]
    u = _np_uniform(42, prob.shape).reshape(b, _H, _W)
    up = np.ones((_H, _NL), np.float32)
    up[:, _L0:_L0 + _W] = u[0]
    up[:, _L1:_L1 + _W] = u[1]

    out = pl.pallas_call(
        _marl_kernel,
        out_shape=jax.ShapeDtypeStruct((1, 1), jnp.float32),
        scratch_shapes=[
            pltpu.VMEM((_H, _NL), jnp.float32),
            pltpu.VMEM((_PR, _NL), jnp.float32),
            pltpu.VMEM((_NSLOT, _PR, _NL), jnp.float32),
            pltpu.VMEM((_NSLOT, _PR, _NL), jnp.float32),
        ],
    )(prob.reshape(b, _H, _W), c.reshape(b, _H, _W), jnp.asarray(up))

    return out.reshape(())


# speedup vs baseline: 2547.6540x; 1.1594x over previous
import numpy as np
import jax
import jax.numpy as jnp
from jax.experimental import pallas as pl
from jax.experimental.pallas import tpu as pltpu

_EPS = 1e-12
_KS = 11
_HALF = _KS // 2
_SIGMA = 2.0
_WS = 0.06
_C1 = (0.01 * 1) ** 2
_C2 = (0.03 * 1) ** 2
_H = 32
_W = 32
_HW = _H * _W
_PAD = 8
_PR = _H + 2 * _PAD
_NL = 128
_L0 = 8
_L1 = 72

_r = np.arange(_KS, dtype=np.float64) - _HALF
_yy, _xx = np.meshgrid(_r, _r, indexing="ij")
_k2 = np.exp(-0.5 * (_xx**2 + _yy**2) / _SIGMA**2)
_k2 = (_k2 / _k2.sum()).astype(np.float32)
_K2 = [[float(_k2[i, j]) for j in range(_KS)] for i in range(_KS)]
_g1 = np.exp(-0.5 * _r**2 / _SIGMA**2)
_g1 = (_g1 / _g1.sum()).astype(np.float32)
_G1 = [float(_g1[i]) for i in range(_KS)]

_G1SQ = [g * g for g in _G1]
_BOX = [1.0] * _KS

_TMAP = {}
for _q1 in range(_HALF + 1):
    for _q2 in range(_q1, _HALF + 1):
        _TMAP[(_q1, _q2)] = len(_TMAP)
_NKV = len(_TMAP)
_GH = [_G1[_HALF + q] for q in range(_HALF + 1)]


def _psqrt(x):
    return x * jax.lax.rsqrt(x)


def _np_threefry2x32(k1, k2, x0, x1):
    rot_a = (13, 15, 26, 6)
    rot_b = (17, 29, 16, 24)

    def rl(x, r):
        return ((x << np.uint32(r)) | (x >> np.uint32(32 - r))).astype(
            np.uint32)

    def rounds(x, rs):
        for r in rs:
            x[0] = (x[0] + x[1]).astype(np.uint32)
            x[1] = x[0] ^ rl(x[1], r)
        return x

    ks = [k1, k2, np.uint32(k1 ^ k2 ^ np.uint32(0x1BD11BDA))]
    x = [(x0 + ks[0]).astype(np.uint32), (x1 + ks[1]).astype(np.uint32)]
    sched = [(rot_a, 1, 2), (rot_b, 2, 0), (rot_a, 0, 1), (rot_b, 1, 2),
             (rot_a, 2, 0)]
    for i, (rs, a, b) in enumerate(sched):
        x = rounds(x, rs)
        x[0] = (x[0] + ks[a]).astype(np.uint32)
        x[1] = (x[1] + ks[b] + np.uint32(i + 1)).astype(np.uint32)
    return x


def _np_uniform(seed, shape):
    n = int(np.prod(shape))
    hi = np.zeros(n, dtype=np.uint32)
    lo = np.arange(n, dtype=np.uint32)
    b = _np_threefry2x32(np.uint32(seed >> 32), np.uint32(seed & 0xFFFFFFFF),
                         hi, lo)
    bits = (b[0] ^ b[1]).astype(np.uint32)
    fb = ((bits >> np.uint32(9)) | np.uint32(0x3F800000)).view(
        np.float32) - np.float32(1.0)
    return np.maximum(np.float32(0.0), fb).reshape(shape)


def _lroll(x, k):
    k %= _NL
    if k == 0:
        return x
    return jnp.concatenate([x[:, -k:], x[:, :-k]], axis=1)


def _marl_kernel(prob_ref, c_ref, u_ref, out_ref, pk, pv, st, std):
    def pack(x0, x1):
        pk[:] = jnp.zeros((_H, _NL), jnp.float32)
        pk[:, _L0:_L0 + _W] = x0
        pk[:, _L1:_L1 + _W] = x1
        return pk[...]

    prob_p = pack(prob_ref[0], prob_ref[1])
    c_p = pack(c_ref[0], c_ref[1])
    ones = jnp.ones((_H, _W), jnp.float32)
    imask = pack(ones, ones)
    h = jnp.where(u_ref[...] < prob_p, 1.0, 0.0)

    def conv(x, g=_G1):
        pv[:] = jnp.zeros((_PR, _NL), jnp.float32)
        pv[_PAD:_PAD + _H, :] = x
        o0 = _PAD - _HALF
        tmp = g[0] * pv[o0:o0 + _H, :]
        for i in range(1, _KS):
            tmp = tmp + g[i] * pv[o0 + i:o0 + i + _H, :]
        out = g[_HALF] * tmp
        for j in range(_KS):
            if j != _HALF:
                out = out + g[j] * _lroll(tmp, _HALF - j)
        return out

    mu_b = conv(h)
    mu_c = conv(c_p)
    hvs_c2 = conv(c_p * c_p)
    hvs_hc = conv(h * c_p)
    c_var = conv(imask * ((c_p - mu_c) * (c_p - mu_c)))

    sig_c = hvs_c2 - mu_c * mu_c
    cc = imask * jnp.clip(2.0 * _psqrt(c_var + _EPS), 0.0, 1.0)
    sig_h_b = mu_b - mu_b * mu_b
    sig_hc_b = hvs_hc - mu_b * mu_c
    l_b = (2.0 * mu_b * mu_c + _C1) / (mu_b * mu_b + mu_c * mu_c + _C1)
    sq_b = _psqrt(jnp.maximum(sig_h_b * sig_c, 0.0) + _EPS)
    cm_b = (2.0 * sq_b + _C2) / (sig_h_b + sig_c + _C2)
    sm_b = (2.0 * sig_hc_b + _C2) / (sq_b + _C2 + _EPS)
    ssim_b = l_b * cm_b * sm_b
    d_b = mu_b - mu_c
    base_map = imask * (_WS * (cc * ssim_b + (1.0 - cc)) - d_b * d_b)

    conv_db2 = 2.0 * conv(imask * d_b)
    k2sum = conv(imask, _G1SQ)
    box_ccs = conv(cc * ssim_b, _BOX)

    rows = slice(_PAD, _PAD + _H)
    mc2c1v = mu_c * mu_c + _C1
    scc2v = sig_c + _C2
    st[:] = jnp.zeros((4 * _NKV, _PR, _NL), jnp.float32)
    for (q1, q2), ti in _TMAP.items():
        kv = float(np.float32(_GH[q1]) * np.float32(_GH[q2]))
        for si, sgn in ((0, 1.0), (1, -1.0)):
            dkv = sgn * kv
            mu = mu_b + dkv
            mu2 = mu * mu
            sig_h = mu - mu2
            mumc = mu * mu_c
            n1 = mumc + mumc + _C1
            d1 = mu2 + mc2c1v
            sq = _psqrt(jnp.maximum(sig_h * sig_c, 0.0) + _EPS)
            d2 = sig_h + scc2v
            n2 = sq + sq + _C2
            d3 = sq + (_C2 + _EPS)
            hcmm = hvs_hc - mumc
            n3b = hcmm + hcmm + _C2
            q_f = (n1 * n2) / ((d1 * d2) * d3)
            ccq = cc * q_f
            st[4 * ti + 2 * si, rows, :] = ccq * n3b
            st[4 * ti + 2 * si + 1, rows, :] = (2.0 * dkv) * ccq

    delta = 1.0 - 2.0 * h
    w = h + delta * prob_p
    mse_acc = delta * conv_db2 + k2sum
    dpos = delta > 0.0

    cssim_acc = jnp.zeros((_H, _NL), jnp.float32)
    for dx in range(_KS):
        qx = abs(dx - _HALF)
        for qy in range(_HALF + 1):
            ti = _TMAP[(min(qy, qx), max(qy, qx))]
            for k in range(4):
                std[4 * qy + k] = _lroll(st[4 * ti + k], _HALF - dx)
        for dy in range(_KS):
            qy = abs(dy - _HALF)
            y0 = _PAD - _HALF + dy
            b0 = 4 * qy
            t0p = std[b0 + 0, y0:y0 + _H, :]
            t1p = std[b0 + 1, y0:y0 + _H, :]
            t0m = std[b0 + 2, y0:y0 + _H, :]
            t1m = std[b0 + 3, y0:y0 + _H, :]
            t0 = jnp.where(dpos, t0p, t0m)
            t1 = jnp.where(dpos, t1p, t1m)
            cssim_acc = cssim_acc + (t0 + t1 * c_p)

    d_r = (_WS * (cssim_acc - box_ccs) - mse_acc) * (1.0 / _HW)
    total = base_map + w * d_r
    t1 = jnp.sum(total, axis=0, keepdims=True)
    out_ref[:] = jnp.sum(t1, axis=1, keepdims=True) * (-1.0 / (2 * _HW))


def kernel(prob, c, z):
    del z
    b = prob.shape[0]
    u = _np_uniform(42, prob.shape).reshape(b, _H, _W)
    up = np.ones((_H, _NL), np.float32)
    up[:, _L0:_L0 + _W] = u[0]
    up[:, _L1:_L1 + _W] = u[1]

    out = pl.pallas_call(
        _marl_kernel,
        out_shape=jax.ShapeDtypeStruct((1, 1), jnp.float32),
        scratch_shapes=[
            pltpu.VMEM((_H, _NL), jnp.float32),
            pltpu.VMEM((_PR, _NL), jnp.float32),
            pltpu.VMEM((4 * _NKV, _PR, _NL), jnp.float32),
            pltpu.VMEM((24, _PR, _NL), jnp.float32),
        ],
    )(prob.reshape(b, _H, _W), c.reshape(b, _H, _W), jnp.asarray(up))

    return out.reshape(())
